# Initial kernel scaffold; baseline (speedup 1.0000x reference)
import jax
import jax.numpy as jnp
from jax.experimental import pallas as pl


def kernel(x_user, x_item, ei_buys, ei_similar, ei_views, Wl_buys, bl_buys, Wr_buys, Wl_similar, bl_similar, Wr_similar, Wl_views, bl_views, Wr_views, W1, b1, W2, b2):
    raise NotImplementedError("write your pallas kernel here")


# speedup vs baseline: 6.1351x; 6.1351x over previous
import jax
import jax.numpy as jnp
from jax import lax
from jax.experimental import pallas as pl
from jax.experimental.pallas import tpu as pltpu
from jax.experimental.pallas import tpu_sc as plsc

N = 10000
D = 128
K = 128
NC = 2
NS = 16
NW = NC * NS
RPT = N // NS
ZR = 25
CZR = 125
CW = 16


def _sc_body(xu, xi, e0, e1, e2, sums, cnts,
             acc_sh, cnt_sh, rows, sidx, didx, ones_r, zb, cb, gsem):
    c = lax.axis_index("c")
    s = lax.axis_index("s")
    wid = s * NC + c
    ch_total = e0.shape[1] // K
    lo = (wid * ch_total) // NW
    hi = ((wid + 1) * ch_total) // NW

    lane = lax.iota(jnp.int32, 16)
    pat = jnp.where(lane == 0, 1.0, 0.0).astype(jnp.float32)
    z16 = jnp.zeros((16,), jnp.float32)

    def fill_ones(i, carry):
        ones_r[i, :] = pat
        return carry
    lax.fori_loop(0, K, fill_ones, 0)

    def fill_zb(i, carry):
        for q in range(D // 16):
            zb[i, pl.ds(q * 16, 16)] = z16
        return carry
    lax.fori_loop(0, ZR, fill_zb, 0)

    def fill_cb(i, carry):
        cb[i, :] = z16
        return carry
    lax.fori_loop(0, CZR, fill_cb, 0)

    for r, (ei, x) in enumerate(((e0, xu), (e1, xi), (e2, xu))):
        for j in range(RPT // ZR):
            pltpu.sync_copy(zb, acc_sh.at[pl.ds(s * RPT + j * ZR, ZR)])
        for j in range(RPT // CZR):
            pltpu.sync_copy(cb, cnt_sh.at[pl.ds(s * RPT + j * CZR, CZR)])
        plsc.subcore_barrier()

        def edge_chunk(ch, carry):
            eb = ch * K
            pltpu.sync_copy(ei.at[0, pl.ds(eb, K)], sidx)
            pltpu.sync_copy(ei.at[1, pl.ds(eb, K)], didx.at[0])
            pltpu.async_copy(x.at[sidx], rows, gsem).wait()
            pltpu.sync_copy(rows, acc_sh.at[didx.at[0]], add=True)
            pltpu.sync_copy(ones_r, cnt_sh.at[didx.at[0]], add=True)
            return carry
        lax.fori_loop(lo, hi, edge_chunk, 0)
        plsc.subcore_barrier()

        pltpu.sync_copy(acc_sh.at[pl.ds(s * RPT, RPT)],
                        sums.at[r, c, pl.ds(s * RPT, RPT)])
        pltpu.sync_copy(cnt_sh.at[pl.ds(s * RPT, RPT)],
                        cnts.at[r, c, pl.ds(s * RPT, RPT)])
        plsc.subcore_barrier()


_sc_segment_sums = pl.kernel(
    _sc_body,
    out_type=(
        jax.ShapeDtypeStruct((3, NC, N, D), jnp.float32),
        jax.ShapeDtypeStruct((3, NC, N, CW), jnp.float32),
    ),
    mesh=plsc.VectorSubcoreMesh(core_axis_name="c", subcore_axis_name="s"),
    compiler_params=pltpu.CompilerParams(use_tc_tiling_on_sc=False),
    scratch_types=[
        pltpu.VMEM_SHARED((N, D), jnp.float32),
        pltpu.VMEM_SHARED((N, CW), jnp.float32),
        pltpu.VMEM((K, D), jnp.float32),
        pltpu.VMEM((K,), jnp.int32),
        pltpu.VMEM((1, K), jnp.int32),
        pltpu.VMEM((K, CW), jnp.float32),
        pltpu.VMEM((ZR, D), jnp.float32),
        pltpu.VMEM((CZR, CW), jnp.float32),
        pltpu.SemaphoreType.DMA,
    ],
)


BL = 1000


def _tc_body(sums, cnts, xi, Wl, bl, Wr, W1, b1, W2, b2, out):
    x_dst = xi[...]
    pre = jnp.zeros((BL, D), jnp.float32)
    for r in range(3):
        ssum = sums[r, 0] + sums[r, 1]
        cnt = cnts[r, 0, :, 0:1] + cnts[r, 1, :, 0:1]
        mean = ssum / jnp.maximum(cnt, 1.0)
        o = lax.dot_general(mean, Wl[r], (((1,), (1,)), ((), ())),
                            preferred_element_type=jnp.float32)
        o = o + bl[r] + lax.dot_general(x_dst, Wr[r], (((1,), (1,)), ((), ())),
                                        preferred_element_type=jnp.float32)
        pre = pre + lax.dot_general(o, W1[:, r * D:(r + 1) * D],
                                    (((1,), (1,)), ((), ())),
                                    preferred_element_type=jnp.float32)
    h = jnp.maximum(pre + b1[...], 0.0)
    out[...] = lax.dot_general(h, W2[...], (((1,), (1,)), ((), ())),
                               preferred_element_type=jnp.float32) + b2[...]


def _tc_tail(sums, cnts, x_item, Wl, bl, Wr, W1, b1, W2, b2):
    grid = N // BL
    return pl.pallas_call(
        _tc_body,
        grid=(grid,),
        in_specs=[
            pl.BlockSpec((3, NC, BL, D), lambda i: (0, 0, i, 0)),
            pl.BlockSpec((3, NC, BL, CW), lambda i: (0, 0, i, 0)),
            pl.BlockSpec((BL, D), lambda i: (i, 0)),
            pl.BlockSpec((3, D, D), lambda i: (0, 0, 0)),
            pl.BlockSpec((3, 1, D), lambda i: (0, 0, 0)),
            pl.BlockSpec((3, D, D), lambda i: (0, 0, 0)),
            pl.BlockSpec((D, 3 * D), lambda i: (0, 0)),
            pl.BlockSpec((1, D), lambda i: (0, 0)),
            pl.BlockSpec((D, D), lambda i: (0, 0)),
            pl.BlockSpec((1, D), lambda i: (0, 0)),
        ],
        out_specs=pl.BlockSpec((BL, D), lambda i: (i, 0)),
        out_shape=jax.ShapeDtypeStruct((N, D), jnp.float32),
    )(sums, cnts, x_item, Wl, bl, Wr, W1, b1, W2, b2)


def kernel(x_user, x_item, ei_buys, ei_similar, ei_views,
           Wl_buys, bl_buys, Wr_buys,
           Wl_similar, bl_similar, Wr_similar,
           Wl_views, bl_views, Wr_views,
           W1, b1, W2, b2):
    sums, cnts = _sc_segment_sums(x_user, x_item, ei_buys, ei_similar, ei_views)
    Wl = jnp.stack([Wl_buys, Wl_similar, Wl_views])
    bl = jnp.stack([bl_buys, bl_similar, bl_views])[:, None, :]
    Wr = jnp.stack([Wr_buys, Wr_similar, Wr_views])
    return _tc_tail(sums, cnts, x_item, Wl, bl, Wr,
                    W1, b1[None, :], W2, b2[None, :])


# speedup vs baseline: 11.1571x; 1.8186x over previous
import jax
import jax.numpy as jnp
from jax import lax
from jax.experimental import pallas as pl
from jax.experimental.pallas import tpu as pltpu
from jax.experimental.pallas import tpu_sc as plsc

N = 10000
D = 128
K = 128
NC = 2
NS = 16
NW = NC * NS
RPT = N // NS
CZR = 125
CW = 16
NB = 8


def _sc_body(xu, xi, e0, e1, e2, sums, cnts,
             acc_sh, cnt_sh, rows0, rows1, sidx_b, didx_b, ones_r, cb,
             gsem0, gsem1, osem):
    c = lax.axis_index("c")
    s = lax.axis_index("s")
    wid = s * NC + c
    ch_total = e0.shape[1]
    base = ch_total // NW
    rem = ch_total % NW
    lo = wid * base
    rows_bufs = (rows0, rows1)
    gsems = (gsem0, gsem1)

    lane = lax.iota(jnp.int32, 16)
    z16 = jnp.zeros((16,), jnp.float32)

    def fill_cb(i, carry):
        cb[i, :] = z16
        return carry
    lax.fori_loop(0, CZR, fill_cb, 0)

    def fill_zero_rows(i, carry):
        for q in range(D // 16):
            rows0[i, pl.ds(q * 16, 16)] = z16
        return carry

    def process_chunks(x, e, start, nchunk):
        pltpu.sync_copy(e.at[0, pl.ds(start, nchunk), :], sidx_b.at[pl.ds(0, nchunk)])
        pltpu.sync_copy(e.at[1, pl.ds(start, nchunk), :], didx_b.at[pl.ds(0, nchunk)])
        descs = [None, None]
        descs[0] = pltpu.async_copy(x.at[sidx_b.at[0]], rows_bufs[0], gsems[0])
        odescs = []
        for j in range(nchunk):
            if j + 1 < nchunk:
                b = (j + 1) % 2
                descs[b] = pltpu.async_copy(x.at[sidx_b.at[j + 1]],
                                            rows_bufs[b], gsems[b])
            descs[j % 2].wait()
            pltpu.sync_copy(rows_bufs[j % 2], acc_sh.at[didx_b.at[j]], add=True)
            odescs.append(pltpu.async_copy(ones_r, cnt_sh.at[didx_b.at[j]],
                                           osem, add=True))
        for od in odescs:
            od.wait()

    for r, (e, x) in enumerate(((e0, xu), (e1, xi), (e2, xu))):
        pat = jnp.where(lane == r, 1.0, 0.0).astype(jnp.float32)

        def fill_ones(i, carry):
            ones_r[i, :] = pat
            return carry
        lax.fori_loop(0, K, fill_ones, 0)

        lax.fori_loop(0, K, fill_zero_rows, 0)
        for j in range(RPT // K):
            pltpu.sync_copy(rows0, acc_sh.at[pl.ds(s * RPT + j * K, K)])
        tail = RPT % K
        if tail:
            pltpu.sync_copy(rows0.at[pl.ds(0, tail)],
                            acc_sh.at[pl.ds(s * RPT + (RPT // K) * K, tail)])
        if r == 0:
            for j in range(RPT // CZR):
                pltpu.sync_copy(cb, cnt_sh.at[pl.ds(s * RPT + j * CZR, CZR)])
        plsc.subcore_barrier()

        def batch_body(bi, carry):
            process_chunks(x, e, lo + bi * NB, NB)
            return carry
        lax.fori_loop(0, base // NB, batch_body, 0)
        if base % NB:
            process_chunks(x, e, lo + (base // NB) * NB, base % NB)

        if rem:
            @pl.when(wid < rem)
            def _():
                process_chunks(x, e, ch_total - rem + wid, 1)

        plsc.subcore_barrier()
        pltpu.sync_copy(acc_sh.at[pl.ds(s * RPT, RPT)],
                        sums.at[r, c, pl.ds(s * RPT, RPT)])
        plsc.subcore_barrier()

    pltpu.sync_copy(cnt_sh.at[pl.ds(s * RPT, RPT)],
                    cnts.at[c, pl.ds(s * RPT, RPT)])


_sc_segment_sums = pl.kernel(
    _sc_body,
    out_type=(
        jax.ShapeDtypeStruct((3, NC, N, D), jnp.float32),
        jax.ShapeDtypeStruct((NC, N, CW), jnp.float32),
    ),
    mesh=plsc.VectorSubcoreMesh(core_axis_name="c", subcore_axis_name="s"),
    compiler_params=pltpu.CompilerParams(use_tc_tiling_on_sc=False),
    scratch_types=[
        pltpu.VMEM_SHARED((N, D), jnp.float32),
        pltpu.VMEM_SHARED((N, CW), jnp.float32),
        pltpu.VMEM((K, D), jnp.float32),
        pltpu.VMEM((K, D), jnp.float32),
        pltpu.VMEM((NB, K), jnp.int32),
        pltpu.VMEM((NB, K), jnp.int32),
        pltpu.VMEM((K, CW), jnp.float32),
        pltpu.VMEM((CZR, CW), jnp.float32),
        pltpu.SemaphoreType.DMA,
        pltpu.SemaphoreType.DMA,
        pltpu.SemaphoreType.DMA,
    ],
)


BL = 1000


def _tc_body(sums, cnts, xi, Wl, bl, Wr, W1, b1, W2, b2, out):
    x_dst = xi[...]
    pre = jnp.zeros((BL, D), jnp.float32)
    for r in range(3):
        ssum = sums[r, 0] + sums[r, 1]
        cnt = cnts[0, :, r:r + 1] + cnts[1, :, r:r + 1]
        mean = ssum / jnp.maximum(cnt, 1.0)
        o = lax.dot_general(mean, Wl[r], (((1,), (1,)), ((), ())),
                            preferred_element_type=jnp.float32)
        o = o + bl[r] + lax.dot_general(x_dst, Wr[r], (((1,), (1,)), ((), ())),
                                        preferred_element_type=jnp.float32)
        pre = pre + lax.dot_general(o, W1[:, r * D:(r + 1) * D],
                                    (((1,), (1,)), ((), ())),
                                    preferred_element_type=jnp.float32)
    h = jnp.maximum(pre + b1[...], 0.0)
    out[...] = lax.dot_general(h, W2[...], (((1,), (1,)), ((), ())),
                               preferred_element_type=jnp.float32) + b2[...]


def _tc_tail(sums, cnts, x_item, Wl, bl, Wr, W1, b1, W2, b2):
    grid = N // BL
    return pl.pallas_call(
        _tc_body,
        grid=(grid,),
        in_specs=[
            pl.BlockSpec((3, NC, BL, D), lambda i: (0, 0, i, 0)),
            pl.BlockSpec((NC, BL, CW), lambda i: (0, i, 0)),
            pl.BlockSpec((BL, D), lambda i: (i, 0)),
            pl.BlockSpec((3, D, D), lambda i: (0, 0, 0)),
            pl.BlockSpec((3, 1, D), lambda i: (0, 0, 0)),
            pl.BlockSpec((3, D, D), lambda i: (0, 0, 0)),
            pl.BlockSpec((D, 3 * D), lambda i: (0, 0)),
            pl.BlockSpec((1, D), lambda i: (0, 0)),
            pl.BlockSpec((D, D), lambda i: (0, 0)),
            pl.BlockSpec((1, D), lambda i: (0, 0)),
        ],
        out_specs=pl.BlockSpec((BL, D), lambda i: (i, 0)),
        out_shape=jax.ShapeDtypeStruct((N, D), jnp.float32),
    )(sums, cnts, x_item, Wl, bl, Wr, W1, b1, W2, b2)


def kernel(x_user, x_item, ei_buys, ei_similar, ei_views,
           Wl_buys, bl_buys, Wr_buys,
           Wl_similar, bl_similar, Wr_similar,
           Wl_views, bl_views, Wr_views,
           W1, b1, W2, b2):
    E = ei_buys.shape[1]
    e0 = ei_buys.reshape(2, E // K, K)
    e1 = ei_similar.reshape(2, E // K, K)
    e2 = ei_views.reshape(2, E // K, K)
    sums, cnts = _sc_segment_sums(x_user, x_item, e0, e1, e2)
    Wl = jnp.stack([Wl_buys, Wl_similar, Wl_views])
    bl = jnp.stack([bl_buys, bl_similar, bl_views])[:, None, :]
    Wr = jnp.stack([Wr_buys, Wr_similar, Wr_views])
    return _tc_tail(sums, cnts, x_item, Wl, bl, Wr,
                    W1, b1[None, :], W2, b2[None, :])


# speedup vs baseline: 11.9731x; 1.0731x over previous
import jax
import jax.numpy as jnp
from jax import lax
from jax.experimental import pallas as pl
from jax.experimental.pallas import tpu as pltpu
from jax.experimental.pallas import tpu_sc as plsc

N = 10000
D = 128
K = 128
NC = 2
NS = 16
NW = NC * NS
RPT = N // NS
CZR = 125
CW = 8
NB = 8


def _sc_body(xu, xi, e0, e1, e2, cpat, sums, cnts,
             acc_sh, cnt_sh, rows0, rows1, sidx_a, didx_a, sidx_b, didx_b,
             ones_r, gsem0, gsem1, ssem0, ssem1, osem, isem_a, isem_b):
    c = lax.axis_index("c")
    s = lax.axis_index("s")
    wid = s * NC + c
    ch_total = e0.shape[1]
    base = ch_total // NW
    rem = ch_total % NW
    lo = wid * base
    rows_bufs = (rows0, rows1)
    gsems = (gsem0, gsem1)
    ssems = (ssem0, ssem1)

    z16 = jnp.zeros((16,), jnp.float32)

    def fill_zero_rows(i, carry):
        for q in range(D // 16):
            rows0[i, pl.ds(q * 16, 16)] = z16
        return carry

    def load_idx(e, start, n, si, di, isem):
        a = pltpu.async_copy(e.at[0, pl.ds(start, n), :], si.at[pl.ds(0, n)], isem)
        b = pltpu.async_copy(e.at[1, pl.ds(start, n), :], di.at[pl.ds(0, n)], isem)
        return a, b

    def wait_idx(e, n, si, di, isem):
        pltpu.make_async_copy(e.at[0, pl.ds(0, n), :], si.at[pl.ds(0, n)], isem).wait()
        pltpu.make_async_copy(e.at[1, pl.ds(0, n), :], di.at[pl.ds(0, n)], isem).wait()

    def process_batch(x, si, di, nchunk):
        gd = [None, None]
        sd = [None, None]
        od = []
        gd[0] = pltpu.async_copy(x.at[si.at[0]], rows_bufs[0], gsems[0])
        for j in range(nchunk):
            b = j % 2
            nb = (j + 1) % 2
            if j + 1 < nchunk:
                if sd[nb] is not None:
                    sd[nb].wait()
                gd[nb] = pltpu.async_copy(x.at[si.at[j + 1]],
                                          rows_bufs[nb], gsems[nb])
            gd[b].wait()
            sd[b] = pltpu.async_copy(rows_bufs[b], acc_sh.at[di.at[j]],
                                     ssems[b], add=True)
            od.append(pltpu.async_copy(ones_r, cnt_sh.at[di.at[j]],
                                       osem, add=True))
        for dsc in sd:
            if dsc is not None:
                dsc.wait()
        for dsc in od:
            dsc.wait()

    for r, (e, x) in enumerate(((e0, xu), (e1, xi), (e2, xu))):
        pltpu.sync_copy(cpat.at[r], ones_r)

        lax.fori_loop(0, K, fill_zero_rows, 0)
        for j in range(RPT // K):
            pltpu.sync_copy(rows0, acc_sh.at[pl.ds(s * RPT + j * K, K)])
        tail = RPT % K
        if tail:
            pltpu.sync_copy(rows0.at[pl.ds(0, tail)],
                            acc_sh.at[pl.ds(s * RPT + (RPT // K) * K, tail)])
        if r == 0:
            pltpu.sync_copy(cpat.at[3], ones_r)
            for j in range(RPT // CZR):
                pltpu.sync_copy(ones_r.at[pl.ds(0, CZR)],
                                cnt_sh.at[pl.ds(s * RPT + j * CZR, CZR)])
            pltpu.sync_copy(cpat.at[r], ones_r)
        plsc.subcore_barrier()

        nfull = base // NB
        npair = nfull // 2
        load_idx(e, lo, NB, sidx_a, didx_a, isem_a)

        def pair_body(k, carry):
            st = lo + (2 * k) * NB
            load_idx(e, st + NB, NB, sidx_b, didx_b, isem_b)
            wait_idx(e, NB, sidx_a, didx_a, isem_a)
            process_batch(x, sidx_a, didx_a, NB)
            load_idx(e, st + 2 * NB, NB, sidx_a, didx_a, isem_a)
            wait_idx(e, NB, sidx_b, didx_b, isem_b)
            process_batch(x, sidx_b, didx_b, NB)
            return carry
        lax.fori_loop(0, npair, pair_body, 0)

        if nfull % 2:
            wait_idx(e, NB, sidx_a, didx_a, isem_a)
            process_batch(x, sidx_a, didx_a, NB)
        btail = base % NB
        if btail:
            ld = load_idx(e, lo + nfull * NB, btail, sidx_b, didx_b, isem_b)
            ld[0].wait()
            ld[1].wait()
            process_batch(x, sidx_b, didx_b, btail)
        if rem:
            @pl.when(wid < rem)
            def _():
                ld = load_idx(e, ch_total - rem + wid, 1, sidx_a, didx_a, isem_a)
                ld[0].wait()
                ld[1].wait()
                process_batch(x, sidx_a, didx_a, 1)

        plsc.subcore_barrier()
        pltpu.sync_copy(acc_sh.at[pl.ds(s * RPT, RPT)],
                        sums.at[r, c, pl.ds(s * RPT, RPT)])
        plsc.subcore_barrier()

    pltpu.sync_copy(cnt_sh.at[pl.ds(s * RPT, RPT)],
                    cnts.at[c, pl.ds(s * RPT, RPT)])


_sc_segment_sums = pl.kernel(
    _sc_body,
    out_type=(
        jax.ShapeDtypeStruct((3, NC, N, D), jnp.float32),
        jax.ShapeDtypeStruct((NC, N, CW), jnp.float32),
    ),
    mesh=plsc.VectorSubcoreMesh(core_axis_name="c", subcore_axis_name="s"),
    compiler_params=pltpu.CompilerParams(use_tc_tiling_on_sc=False),
    scratch_types=[
        pltpu.VMEM_SHARED((N, D), jnp.float32),
        pltpu.VMEM_SHARED((N, CW), jnp.float32),
        pltpu.VMEM((K, D), jnp.float32),
        pltpu.VMEM((K, D), jnp.float32),
        pltpu.VMEM((NB, K), jnp.int32),
        pltpu.VMEM((NB, K), jnp.int32),
        pltpu.VMEM((NB, K), jnp.int32),
        pltpu.VMEM((NB, K), jnp.int32),
        pltpu.VMEM((K, CW), jnp.float32),
        pltpu.SemaphoreType.DMA,
        pltpu.SemaphoreType.DMA,
        pltpu.SemaphoreType.DMA,
        pltpu.SemaphoreType.DMA,
        pltpu.SemaphoreType.DMA,
        pltpu.SemaphoreType.DMA,
        pltpu.SemaphoreType.DMA,
    ],
)


BL = 1000


def _tc_body(sums, cnts, xi, Wl, bl, Wr, W1, b1, W2, b2, out):
    x_dst = xi[...]
    pre = jnp.zeros((BL, D), jnp.float32)
    for r in range(3):
        ssum = sums[r, 0] + sums[r, 1]
        cnt = cnts[0, :, r:r + 1] + cnts[1, :, r:r + 1]
        mean = ssum / jnp.maximum(cnt, 1.0)
        o = lax.dot_general(mean, Wl[r], (((1,), (1,)), ((), ())),
                            preferred_element_type=jnp.float32)
        o = o + bl[r] + lax.dot_general(x_dst, Wr[r], (((1,), (1,)), ((), ())),
                                        preferred_element_type=jnp.float32)
        pre = pre + lax.dot_general(o, W1[:, r * D:(r + 1) * D],
                                    (((1,), (1,)), ((), ())),
                                    preferred_element_type=jnp.float32)
    h = jnp.maximum(pre + b1[...], 0.0)
    out[...] = lax.dot_general(h, W2[...], (((1,), (1,)), ((), ())),
                               preferred_element_type=jnp.float32) + b2[...]


def _tc_tail(sums, cnts, x_item, Wl, bl, Wr, W1, b1, W2, b2):
    grid = N // BL
    return pl.pallas_call(
        _tc_body,
        grid=(grid,),
        in_specs=[
            pl.BlockSpec((3, NC, BL, D), lambda i: (0, 0, i, 0)),
            pl.BlockSpec((NC, BL, CW), lambda i: (0, i, 0)),
            pl.BlockSpec((BL, D), lambda i: (i, 0)),
            pl.BlockSpec((3, D, D), lambda i: (0, 0, 0)),
            pl.BlockSpec((3, 1, D), lambda i: (0, 0, 0)),
            pl.BlockSpec((3, D, D), lambda i: (0, 0, 0)),
            pl.BlockSpec((D, 3 * D), lambda i: (0, 0)),
            pl.BlockSpec((1, D), lambda i: (0, 0)),
            pl.BlockSpec((D, D), lambda i: (0, 0)),
            pl.BlockSpec((1, D), lambda i: (0, 0)),
        ],
        out_specs=pl.BlockSpec((BL, D), lambda i: (i, 0)),
        out_shape=jax.ShapeDtypeStruct((N, D), jnp.float32),
    )(sums, cnts, x_item, Wl, bl, Wr, W1, b1, W2, b2)


def kernel(x_user, x_item, ei_buys, ei_similar, ei_views,
           Wl_buys, bl_buys, Wr_buys,
           Wl_similar, bl_similar, Wr_similar,
           Wl_views, bl_views, Wr_views,
           W1, b1, W2, b2):
    E = ei_buys.shape[1]
    e0 = ei_buys.reshape(2, E // K, K)
    e1 = ei_similar.reshape(2, E // K, K)
    e2 = ei_views.reshape(2, E // K, K)
    cpat = (jnp.array([0, 1, 2, -1])[:, None, None] == jnp.arange(CW)[None, None, :])
    cpat = jnp.broadcast_to(cpat, (4, K, CW)).astype(jnp.float32)
    sums, cnts = _sc_segment_sums(x_user, x_item, e0, e1, e2, cpat)
    Wl = jnp.stack([Wl_buys, Wl_similar, Wl_views])
    bl = jnp.stack([bl_buys, bl_similar, bl_views])[:, None, :]
    Wr = jnp.stack([Wr_buys, Wr_similar, Wr_views])
    return _tc_tail(sums, cnts, x_item, Wl, bl, Wr,
                    W1, b1[None, :], W2, b2[None, :])


# speedup vs baseline: 12.7017x; 1.0609x over previous
import jax
import jax.numpy as jnp
from jax import lax
from jax.experimental import pallas as pl
from jax.experimental.pallas import tpu as pltpu
from jax.experimental.pallas import tpu_sc as plsc

N = 10000
D = 128
K = 128
NC = 2
NS = 16
NW = NC * NS
RPT = N // NS
CZR = 125
CW = 8
NB = 8


def _sc_body(xu, xi, e0, e1, e2, cpat, sums, cnts,
             acc_sh, cnt_sh, rows0, rows1, sidx_a, didx_a, sidx_b, didx_b,
             sidx_c, didx_c, sidx_d, didx_d,
             ones_r, gsem0, gsem1, ssem0, ssem1, osem,
             isem_a, isem_b, isem_c, isem_d):
    c = lax.axis_index("c")
    s = lax.axis_index("s")
    wid = s * NC + c
    ch_total = e0.shape[1]
    base = ch_total // NW
    rem = ch_total % NW
    lo = wid * base
    rows_bufs = (rows0, rows1)
    gsems = (gsem0, gsem1)
    ssems = (ssem0, ssem1)

    z16 = jnp.zeros((16,), jnp.float32)

    def fill_zero_rows(i, carry):
        for q in range(D // 16):
            rows0[i, pl.ds(q * 16, 16)] = z16
        return carry

    def load_idx(e, start, n, si, di, isem):
        a = pltpu.async_copy(e.at[0, pl.ds(start, n), :], si.at[pl.ds(0, n)], isem)
        b = pltpu.async_copy(e.at[1, pl.ds(start, n), :], di.at[pl.ds(0, n)], isem)
        return a, b

    def wait_idx(e, n, si, di, isem):
        pltpu.make_async_copy(e.at[0, pl.ds(0, n), :], si.at[pl.ds(0, n)], isem).wait()
        pltpu.make_async_copy(e.at[1, pl.ds(0, n), :], di.at[pl.ds(0, n)], isem).wait()

    def process_batch(x, chunks):
        nchunk = len(chunks)
        gd = [None, None]
        sd = [None, None]
        od = []
        gd[0] = pltpu.async_copy(x.at[chunks[0][0]], rows_bufs[0], gsems[0])
        for j in range(nchunk):
            b = j % 2
            nb = (j + 1) % 2
            if j + 1 < nchunk:
                if sd[nb] is not None:
                    sd[nb].wait()
                gd[nb] = pltpu.async_copy(x.at[chunks[j + 1][0]],
                                          rows_bufs[nb], gsems[nb])
            gd[b].wait()
            sd[b] = pltpu.async_copy(rows_bufs[b], acc_sh.at[chunks[j][1]],
                                     ssems[b], add=True)
            od.append(pltpu.async_copy(ones_r, cnt_sh.at[chunks[j][1]],
                                       osem, add=True))
        for dsc in sd:
            if dsc is not None:
                dsc.wait()
        for dsc in od:
            dsc.wait()

    def rows_of(si, di, n):
        return [(si.at[j], di.at[j]) for j in range(n)]

    bufA = (sidx_a, didx_a, isem_a)
    bufB = (sidx_b, didx_b, isem_b)
    bufC = (sidx_c, didx_c, isem_c)
    bufD = (sidx_d, didx_d, isem_d)

    for r, (e, x) in enumerate(((e0, xu), (e1, xi), (e2, xu))):
        load_idx(e, lo, NB, *bufA)
        load_idx(e, lo + NB, NB, *bufB)
        pltpu.sync_copy(cpat.at[r], ones_r)

        lax.fori_loop(0, K, fill_zero_rows, 0)
        for j in range(RPT // K):
            pltpu.sync_copy(rows0, acc_sh.at[pl.ds(s * RPT + j * K, K)])
        tail = RPT % K
        if tail:
            pltpu.sync_copy(rows0.at[pl.ds(0, tail)],
                            acc_sh.at[pl.ds(s * RPT + (RPT // K) * K, tail)])
        if r == 0:
            pltpu.sync_copy(cpat.at[3], ones_r)
            for j in range(RPT // CZR):
                pltpu.sync_copy(ones_r.at[pl.ds(0, CZR)],
                                cnt_sh.at[pl.ds(s * RPT + j * CZR, CZR)])
            pltpu.sync_copy(cpat.at[r], ones_r)
        plsc.subcore_barrier()

        nquad = base // (4 * NB)

        cap = ch_total - NB

        def quad_body(k, carry):
            st = lo + k * (4 * NB)
            load_idx(e, jnp.minimum(st + 2 * NB, cap), NB, *bufC)
            load_idx(e, jnp.minimum(st + 3 * NB, cap), NB, *bufD)
            wait_idx(e, NB, *bufA)
            wait_idx(e, NB, *bufB)
            process_batch(x, rows_of(sidx_a, didx_a, NB)
                          + rows_of(sidx_b, didx_b, NB))
            load_idx(e, jnp.minimum(st + 4 * NB, cap), NB, *bufA)
            load_idx(e, jnp.minimum(st + 5 * NB, cap), NB, *bufB)
            wait_idx(e, NB, *bufC)
            wait_idx(e, NB, *bufD)
            process_batch(x, rows_of(sidx_c, didx_c, NB)
                          + rows_of(sidx_d, didx_d, NB))
            return carry
        lax.fori_loop(0, nquad, quad_body, 0)

        left = base - nquad * 4 * NB
        st0 = lo + nquad * 4 * NB
        wait_idx(e, NB, *bufA)
        wait_idx(e, NB, *bufB)
        pos = 0
        first = True
        while left > 0:
            na = min(left, NB)
            nb2 = min(left - na, NB)
            if first:
                p, q = bufA, bufB
            else:
                p, q = bufC, bufD
                load_idx(e, st0 + pos, na, *p)
                if nb2:
                    load_idx(e, st0 + pos + na, nb2, *q)
                wait_idx(e, na, *p)
                if nb2:
                    wait_idx(e, nb2, *q)
            process_batch(x, rows_of(p[0], p[1], na)
                          + rows_of(q[0], q[1], nb2))
            pos += na + nb2
            left -= na + nb2
            first = False
        if rem:
            @pl.when(wid < rem)
            def _():
                ld = load_idx(e, ch_total - rem + wid, 1, *bufC)
                ld[0].wait()
                ld[1].wait()
                process_batch(x, rows_of(sidx_c, didx_c, 1))

        plsc.subcore_barrier()
        pltpu.sync_copy(acc_sh.at[pl.ds(s * RPT, RPT)],
                        sums.at[r, c, pl.ds(s * RPT, RPT)])
        plsc.subcore_barrier()

    pltpu.sync_copy(cnt_sh.at[pl.ds(s * RPT, RPT)],
                    cnts.at[c, pl.ds(s * RPT, RPT)])


_sc_segment_sums = pl.kernel(
    _sc_body,
    out_type=(
        jax.ShapeDtypeStruct((3, NC, N, D), jnp.float32),
        jax.ShapeDtypeStruct((NC, N, CW), jnp.float32),
    ),
    mesh=plsc.VectorSubcoreMesh(core_axis_name="c", subcore_axis_name="s"),
    compiler_params=pltpu.CompilerParams(use_tc_tiling_on_sc=False),
    scratch_types=[
        pltpu.VMEM_SHARED((N, D), jnp.float32),
        pltpu.VMEM_SHARED((N, CW), jnp.float32),
        pltpu.VMEM((K, D), jnp.float32),
        pltpu.VMEM((K, D), jnp.float32),
        pltpu.VMEM((NB, K), jnp.int32),
        pltpu.VMEM((NB, K), jnp.int32),
        pltpu.VMEM((NB, K), jnp.int32),
        pltpu.VMEM((NB, K), jnp.int32),
        pltpu.VMEM((NB, K), jnp.int32),
        pltpu.VMEM((NB, K), jnp.int32),
        pltpu.VMEM((NB, K), jnp.int32),
        pltpu.VMEM((NB, K), jnp.int32),
        pltpu.VMEM((K, CW), jnp.float32),
        pltpu.SemaphoreType.DMA,
        pltpu.SemaphoreType.DMA,
        pltpu.SemaphoreType.DMA,
        pltpu.SemaphoreType.DMA,
        pltpu.SemaphoreType.DMA,
        pltpu.SemaphoreType.DMA,
        pltpu.SemaphoreType.DMA,
        pltpu.SemaphoreType.DMA,
        pltpu.SemaphoreType.DMA,
    ],
)


BL = 1000


def _tc_body(sums, cnts, xi, Wl, bl, Wr, W1, b1, W2, b2, out):
    x_dst = xi[...]
    pre = jnp.zeros((BL, D), jnp.float32)
    for r in range(3):
        ssum = sums[r, 0] + sums[r, 1]
        cnt = cnts[0, :, r:r + 1] + cnts[1, :, r:r + 1]
        mean = ssum / jnp.maximum(cnt, 1.0)
        o = lax.dot_general(mean, Wl[r], (((1,), (1,)), ((), ())),
                            preferred_element_type=jnp.float32)
        o = o + bl[r] + lax.dot_general(x_dst, Wr[r], (((1,), (1,)), ((), ())),
                                        preferred_element_type=jnp.float32)
        pre = pre + lax.dot_general(o, W1[:, r * D:(r + 1) * D],
                                    (((1,), (1,)), ((), ())),
                                    preferred_element_type=jnp.float32)
    h = jnp.maximum(pre + b1[...], 0.0)
    out[...] = lax.dot_general(h, W2[...], (((1,), (1,)), ((), ())),
                               preferred_element_type=jnp.float32) + b2[...]


def _tc_tail(sums, cnts, x_item, Wl, bl, Wr, W1, b1, W2, b2):
    grid = N // BL
    return pl.pallas_call(
        _tc_body,
        grid=(grid,),
        in_specs=[
            pl.BlockSpec((3, NC, BL, D), lambda i: (0, 0, i, 0)),
            pl.BlockSpec((NC, BL, CW), lambda i: (0, i, 0)),
            pl.BlockSpec((BL, D), lambda i: (i, 0)),
            pl.BlockSpec((3, D, D), lambda i: (0, 0, 0)),
            pl.BlockSpec((3, 1, D), lambda i: (0, 0, 0)),
            pl.BlockSpec((3, D, D), lambda i: (0, 0, 0)),
            pl.BlockSpec((D, 3 * D), lambda i: (0, 0)),
            pl.BlockSpec((1, D), lambda i: (0, 0)),
            pl.BlockSpec((D, D), lambda i: (0, 0)),
            pl.BlockSpec((1, D), lambda i: (0, 0)),
        ],
        out_specs=pl.BlockSpec((BL, D), lambda i: (i, 0)),
        out_shape=jax.ShapeDtypeStruct((N, D), jnp.float32),
    )(sums, cnts, x_item, Wl, bl, Wr, W1, b1, W2, b2)


def kernel(x_user, x_item, ei_buys, ei_similar, ei_views,
           Wl_buys, bl_buys, Wr_buys,
           Wl_similar, bl_similar, Wr_similar,
           Wl_views, bl_views, Wr_views,
           W1, b1, W2, b2):
    E = ei_buys.shape[1]
    e0 = ei_buys.reshape(2, E // K, K)
    e1 = ei_similar.reshape(2, E // K, K)
    e2 = ei_views.reshape(2, E // K, K)
    cpat = (jnp.array([0, 1, 2, -1])[:, None, None] == jnp.arange(CW)[None, None, :])
    cpat = jnp.broadcast_to(cpat, (4, K, CW)).astype(jnp.float32)
    sums, cnts = _sc_segment_sums(x_user, x_item, e0, e1, e2, cpat)
    Wl = jnp.stack([Wl_buys, Wl_similar, Wl_views])
    bl = jnp.stack([bl_buys, bl_similar, bl_views])[:, None, :]
    Wr = jnp.stack([Wr_buys, Wr_similar, Wr_views])
    return _tc_tail(sums, cnts, x_item, Wl, bl, Wr,
                    W1, b1[None, :], W2, b2[None, :])


# speedup vs baseline: 12.7185x; 1.0013x over previous
import jax
import jax.numpy as jnp
from jax import lax
from jax.experimental import pallas as pl
from jax.experimental.pallas import tpu as pltpu
from jax.experimental.pallas import tpu_sc as plsc

N = 10000
D = 128
K = 128
NC = 2
NS = 16
NW = NC * NS
RPT = N // NS
CZR = 125
CW = 8
NB = 8


def _sc_body(xu, xi, e0, e1, e2, cpat, sums, cnts,
             acc_sh, cnt_sh, rows0, rows1, sidx_a, didx_a, sidx_b, didx_b,
             sidx_c, didx_c, sidx_d, didx_d,
             ones_r, gsem0, gsem1, ssem0, ssem1, osem,
             isem_a, isem_b, isem_c, isem_d, dsem):
    c = lax.axis_index("c")
    s = lax.axis_index("s")
    wid = s * NC + c
    ch_total = e0.shape[1]
    base = ch_total // NW
    rem = ch_total % NW
    lo = wid * base
    rows_bufs = (rows0, rows1)
    gsems = (gsem0, gsem1)
    ssems = (ssem0, ssem1)

    z16 = jnp.zeros((16,), jnp.float32)

    def fill_zero_rows(i, carry):
        for q in range(D // 16):
            rows0[i, pl.ds(q * 16, 16)] = z16
        return carry

    def load_idx(e, start, n, si, di, isem):
        a = pltpu.async_copy(e.at[0, pl.ds(start, n), :], si.at[pl.ds(0, n)], isem)
        b = pltpu.async_copy(e.at[1, pl.ds(start, n), :], di.at[pl.ds(0, n)], isem)
        return a, b

    def wait_idx(e, n, si, di, isem):
        pltpu.make_async_copy(e.at[0, pl.ds(0, n), :], si.at[pl.ds(0, n)], isem).wait()
        pltpu.make_async_copy(e.at[1, pl.ds(0, n), :], di.at[pl.ds(0, n)], isem).wait()

    def process_batch(x, chunks):
        nchunk = len(chunks)
        gd = [None, None]
        sd = [None, None]
        od = []
        gd[0] = pltpu.async_copy(x.at[chunks[0][0]], rows_bufs[0], gsems[0])
        for j in range(nchunk):
            b = j % 2
            nb = (j + 1) % 2
            if j + 1 < nchunk:
                if sd[nb] is not None:
                    sd[nb].wait()
                gd[nb] = pltpu.async_copy(x.at[chunks[j + 1][0]],
                                          rows_bufs[nb], gsems[nb])
            gd[b].wait()
            sd[b] = pltpu.async_copy(rows_bufs[b], acc_sh.at[chunks[j][1]],
                                     ssems[b], add=True)
            od.append(pltpu.async_copy(ones_r, cnt_sh.at[chunks[j][1]],
                                       osem, add=True))
        for dsc in sd:
            if dsc is not None:
                dsc.wait()
        for dsc in od:
            dsc.wait()

    def rows_of(si, di, n):
        return [(si.at[j], di.at[j]) for j in range(n)]

    bufA = (sidx_a, didx_a, isem_a)
    bufB = (sidx_b, didx_b, isem_b)
    bufC = (sidx_c, didx_c, isem_c)
    bufD = (sidx_d, didx_d, isem_d)

    def zero_acc_stripe():
        for j in range(RPT // K):
            pltpu.sync_copy(rows0, acc_sh.at[pl.ds(s * RPT + j * K, K)])
        tail = RPT % K
        if tail:
            pltpu.sync_copy(rows0.at[pl.ds(0, tail)],
                            acc_sh.at[pl.ds(s * RPT + (RPT // K) * K, tail)])

    rels = ((e0, xu), (e1, xi), (e2, xu))

    load_idx(rels[0][0], lo, NB, *bufA)
    load_idx(rels[0][0], lo + NB, NB, *bufB)
    lax.fori_loop(0, K, fill_zero_rows, 0)
    zero_acc_stripe()
    pltpu.sync_copy(cpat.at[3], ones_r)
    for j in range(RPT // CZR):
        pltpu.sync_copy(ones_r.at[pl.ds(0, CZR)],
                        cnt_sh.at[pl.ds(s * RPT + j * CZR, CZR)])
    plsc.subcore_barrier()

    for r, (e, x) in enumerate(rels):
        pltpu.sync_copy(cpat.at[r], ones_r)

        nquad = base // (4 * NB)

        cap = ch_total - NB

        def quad_body(k, carry):
            st = lo + k * (4 * NB)
            load_idx(e, jnp.minimum(st + 2 * NB, cap), NB, *bufC)
            load_idx(e, jnp.minimum(st + 3 * NB, cap), NB, *bufD)
            wait_idx(e, NB, *bufA)
            wait_idx(e, NB, *bufB)
            process_batch(x, rows_of(sidx_a, didx_a, NB)
                          + rows_of(sidx_b, didx_b, NB))
            load_idx(e, jnp.minimum(st + 4 * NB, cap), NB, *bufA)
            load_idx(e, jnp.minimum(st + 5 * NB, cap), NB, *bufB)
            wait_idx(e, NB, *bufC)
            wait_idx(e, NB, *bufD)
            process_batch(x, rows_of(sidx_c, didx_c, NB)
                          + rows_of(sidx_d, didx_d, NB))
            return carry
        lax.fori_loop(0, nquad, quad_body, 0)

        left = base - nquad * 4 * NB
        st0 = lo + nquad * 4 * NB
        wait_idx(e, NB, *bufA)
        wait_idx(e, NB, *bufB)
        pos = 0
        first = True
        while left > 0:
            na = min(left, NB)
            nb2 = min(left - na, NB)
            if first:
                p, q = bufA, bufB
            else:
                p, q = bufC, bufD
                load_idx(e, st0 + pos, na, *p)
                if nb2:
                    load_idx(e, st0 + pos + na, nb2, *q)
                wait_idx(e, na, *p)
                if nb2:
                    wait_idx(e, nb2, *q)
            process_batch(x, rows_of(p[0], p[1], na)
                          + rows_of(q[0], q[1], nb2))
            pos += na + nb2
            left -= na + nb2
            first = False
        if rem:
            @pl.when(wid < rem)
            def _():
                ld = load_idx(e, ch_total - rem + wid, 1, *bufC)
                ld[0].wait()
                ld[1].wait()
                process_batch(x, rows_of(sidx_c, didx_c, 1))

        if r < 2:
            load_idx(rels[r + 1][0], lo, NB, *bufA)
            load_idx(rels[r + 1][0], lo + NB, NB, *bufB)
        plsc.subcore_barrier()

        dd = pltpu.async_copy(acc_sh.at[pl.ds(s * RPT, RPT)],
                              sums.at[r, c, pl.ds(s * RPT, RPT)], dsem)
        if r == 2:
            cd = pltpu.async_copy(cnt_sh.at[pl.ds(s * RPT, RPT)],
                                  cnts.at[c, pl.ds(s * RPT, RPT)], dsem)
        else:
            lax.fori_loop(0, K, fill_zero_rows, 0)
        dd.wait()
        if r == 2:
            cd.wait()
        else:
            zero_acc_stripe()
            plsc.subcore_barrier()


_sc_segment_sums = pl.kernel(
    _sc_body,
    out_type=(
        jax.ShapeDtypeStruct((3, NC, N, D), jnp.float32),
        jax.ShapeDtypeStruct((NC, N, CW), jnp.float32),
    ),
    mesh=plsc.VectorSubcoreMesh(core_axis_name="c", subcore_axis_name="s"),
    compiler_params=pltpu.CompilerParams(use_tc_tiling_on_sc=False),
    scratch_types=[
        pltpu.VMEM_SHARED((N, D), jnp.float32),
        pltpu.VMEM_SHARED((N, CW), jnp.float32),
        pltpu.VMEM((K, D), jnp.float32),
        pltpu.VMEM((K, D), jnp.float32),
        pltpu.VMEM((NB, K), jnp.int32),
        pltpu.VMEM((NB, K), jnp.int32),
        pltpu.VMEM((NB, K), jnp.int32),
        pltpu.VMEM((NB, K), jnp.int32),
        pltpu.VMEM((NB, K), jnp.int32),
        pltpu.VMEM((NB, K), jnp.int32),
        pltpu.VMEM((NB, K), jnp.int32),
        pltpu.VMEM((NB, K), jnp.int32),
        pltpu.VMEM((K, CW), jnp.float32),
        pltpu.SemaphoreType.DMA,
        pltpu.SemaphoreType.DMA,
        pltpu.SemaphoreType.DMA,
        pltpu.SemaphoreType.DMA,
        pltpu.SemaphoreType.DMA,
        pltpu.SemaphoreType.DMA,
        pltpu.SemaphoreType.DMA,
        pltpu.SemaphoreType.DMA,
        pltpu.SemaphoreType.DMA,
        pltpu.SemaphoreType.DMA,
    ],
)


BL = 1000


def _tc_body(sums, cnts, xi, Wl, bl, Wr, W1, b1, W2, b2, out):
    x_dst = xi[...]
    pre = jnp.zeros((BL, D), jnp.float32)
    for r in range(3):
        ssum = sums[r, 0] + sums[r, 1]
        cnt = cnts[0, :, r:r + 1] + cnts[1, :, r:r + 1]
        mean = ssum / jnp.maximum(cnt, 1.0)
        o = lax.dot_general(mean, Wl[r], (((1,), (1,)), ((), ())),
                            preferred_element_type=jnp.float32)
        o = o + bl[r] + lax.dot_general(x_dst, Wr[r], (((1,), (1,)), ((), ())),
                                        preferred_element_type=jnp.float32)
        pre = pre + lax.dot_general(o, W1[:, r * D:(r + 1) * D],
                                    (((1,), (1,)), ((), ())),
                                    preferred_element_type=jnp.float32)
    h = jnp.maximum(pre + b1[...], 0.0)
    out[...] = lax.dot_general(h, W2[...], (((1,), (1,)), ((), ())),
                               preferred_element_type=jnp.float32) + b2[...]


def _tc_tail(sums, cnts, x_item, Wl, bl, Wr, W1, b1, W2, b2):
    grid = N // BL
    return pl.pallas_call(
        _tc_body,
        grid=(grid,),
        in_specs=[
            pl.BlockSpec((3, NC, BL, D), lambda i: (0, 0, i, 0)),
            pl.BlockSpec((NC, BL, CW), lambda i: (0, i, 0)),
            pl.BlockSpec((BL, D), lambda i: (i, 0)),
            pl.BlockSpec((3, D, D), lambda i: (0, 0, 0)),
            pl.BlockSpec((3, 1, D), lambda i: (0, 0, 0)),
            pl.BlockSpec((3, D, D), lambda i: (0, 0, 0)),
            pl.BlockSpec((D, 3 * D), lambda i: (0, 0)),
            pl.BlockSpec((1, D), lambda i: (0, 0)),
            pl.BlockSpec((D, D), lambda i: (0, 0)),
            pl.BlockSpec((1, D), lambda i: (0, 0)),
        ],
        out_specs=pl.BlockSpec((BL, D), lambda i: (i, 0)),
        out_shape=jax.ShapeDtypeStruct((N, D), jnp.float32),
    )(sums, cnts, x_item, Wl, bl, Wr, W1, b1, W2, b2)


def kernel(x_user, x_item, ei_buys, ei_similar, ei_views,
           Wl_buys, bl_buys, Wr_buys,
           Wl_similar, bl_similar, Wr_similar,
           Wl_views, bl_views, Wr_views,
           W1, b1, W2, b2):
    E = ei_buys.shape[1]
    e0 = ei_buys.reshape(2, E // K, K)
    e1 = ei_similar.reshape(2, E // K, K)
    e2 = ei_views.reshape(2, E // K, K)
    cpat = (jnp.array([0, 1, 2, -1])[:, None, None] == jnp.arange(CW)[None, None, :])
    cpat = jnp.broadcast_to(cpat, (4, K, CW)).astype(jnp.float32)
    sums, cnts = _sc_segment_sums(x_user, x_item, e0, e1, e2, cpat)
    Wl = jnp.stack([Wl_buys, Wl_similar, Wl_views])
    bl = jnp.stack([bl_buys, bl_similar, bl_views])[:, None, :]
    Wr = jnp.stack([Wr_buys, Wr_similar, Wr_views])
    return _tc_tail(sums, cnts, x_item, Wl, bl, Wr,
                    W1, b1[None, :], W2, b2[None, :])


# speedup vs baseline: 13.6323x; 1.0718x over previous
import jax
import jax.numpy as jnp
from jax import lax
from jax.experimental import pallas as pl
from jax.experimental.pallas import tpu as pltpu
from jax.experimental.pallas import tpu_sc as plsc

N = 10000
D = 128
CHK = 64
NC = 2
NS = 16
NW = NC * NS
RPT = N // NS
CZR = 25
CW = 8
NB = 16
NBUF = 4
LOOK = 2


def _sc_body(xu, xi, e0, e1, e2, cpat, sums, cnts,
             acc_sh, cnt_sh, rows0, rows1, rows2, rows3,
             sidx_a, didx_a, sidx_b, didx_b,
             sidx_c, didx_c, sidx_d, didx_d,
             ones_r, gsem0, gsem1, gsem2, gsem3,
             ssem0, ssem1, ssem2, ssem3, osem,
             isem_a, isem_b, isem_c, isem_d, dsem):
    c = lax.axis_index("c")
    s = lax.axis_index("s")
    wid = s * NC + c
    ch_total = e0.shape[1]
    base = ch_total // NW
    rem = ch_total % NW
    lo = wid * base
    rows_bufs = (rows0, rows1, rows2, rows3)
    gsems = (gsem0, gsem1, gsem2, gsem3)
    ssems = (ssem0, ssem1, ssem2, ssem3)

    z16 = jnp.zeros((16,), jnp.float32)

    def fill_zero_rows(i, carry):
        for q in range(D // 16):
            rows0[i, pl.ds(q * 16, 16)] = z16
        return carry

    def load_idx(e, start, n, si, di, isem):
        a = pltpu.async_copy(e.at[0, pl.ds(start, n), :], si.at[pl.ds(0, n)], isem)
        b = pltpu.async_copy(e.at[1, pl.ds(start, n), :], di.at[pl.ds(0, n)], isem)
        return a, b

    def wait_idx(e, n, si, di, isem):
        pltpu.make_async_copy(e.at[0, pl.ds(0, n), :], si.at[pl.ds(0, n)], isem).wait()
        pltpu.make_async_copy(e.at[1, pl.ds(0, n), :], di.at[pl.ds(0, n)], isem).wait()

    def process_batch(x, chunks):
        nchunk = len(chunks)
        gd = [None] * NBUF
        sd = [None] * NBUF
        od = []
        for t in range(min(LOOK, nchunk)):
            gd[t] = pltpu.async_copy(x.at[chunks[t][0]], rows_bufs[t], gsems[t])
        for j in range(nchunk):
            t = j + LOOK
            if t < nchunk:
                tb = t % NBUF
                if sd[tb] is not None:
                    sd[tb].wait()
                gd[tb] = pltpu.async_copy(x.at[chunks[t][0]],
                                          rows_bufs[tb], gsems[tb])
            b = j % NBUF
            gd[b].wait()
            sd[b] = pltpu.async_copy(rows_bufs[b], acc_sh.at[chunks[j][1]],
                                     ssems[b], add=True)
            od.append(pltpu.async_copy(ones_r, cnt_sh.at[chunks[j][1]],
                                       osem, add=True))
        for dsc in sd:
            if dsc is not None:
                dsc.wait()
        for dsc in od:
            dsc.wait()

    def rows_of(si, di, n):
        return [(si.at[j], di.at[j]) for j in range(n)]

    bufA = (sidx_a, didx_a, isem_a)
    bufB = (sidx_b, didx_b, isem_b)
    bufC = (sidx_c, didx_c, isem_c)
    bufD = (sidx_d, didx_d, isem_d)

    def zero_acc_stripe():
        for j in range(RPT // CHK):
            pltpu.sync_copy(rows0, acc_sh.at[pl.ds(s * RPT + j * CHK, CHK)])
        tail = RPT % CHK
        if tail:
            pltpu.sync_copy(rows0.at[pl.ds(0, tail)],
                            acc_sh.at[pl.ds(s * RPT + (RPT // CHK) * CHK, tail)])

    rels = ((e0, xu), (e1, xi), (e2, xu))

    load_idx(rels[0][0], lo, NB, *bufA)
    load_idx(rels[0][0], lo + NB, NB, *bufB)
    lax.fori_loop(0, CHK, fill_zero_rows, 0)
    zero_acc_stripe()
    pltpu.sync_copy(cpat.at[3], ones_r)
    for j in range(RPT // CZR):
        pltpu.sync_copy(ones_r.at[pl.ds(0, CZR)],
                        cnt_sh.at[pl.ds(s * RPT + j * CZR, CZR)])
    plsc.subcore_barrier()

    for r, (e, x) in enumerate(rels):
        pltpu.sync_copy(cpat.at[r], ones_r)

        nquad = base // (4 * NB)

        cap = ch_total - NB

        def quad_body(k, carry):
            st = lo + k * (4 * NB)
            load_idx(e, jnp.minimum(st + 2 * NB, cap), NB, *bufC)
            load_idx(e, jnp.minimum(st + 3 * NB, cap), NB, *bufD)
            wait_idx(e, NB, *bufA)
            wait_idx(e, NB, *bufB)
            process_batch(x, rows_of(sidx_a, didx_a, NB)
                          + rows_of(sidx_b, didx_b, NB))
            load_idx(e, jnp.minimum(st + 4 * NB, cap), NB, *bufA)
            load_idx(e, jnp.minimum(st + 5 * NB, cap), NB, *bufB)
            wait_idx(e, NB, *bufC)
            wait_idx(e, NB, *bufD)
            process_batch(x, rows_of(sidx_c, didx_c, NB)
                          + rows_of(sidx_d, didx_d, NB))
            return carry
        lax.fori_loop(0, nquad, quad_body, 0)

        left = base - nquad * 4 * NB
        st0 = lo + nquad * 4 * NB
        wait_idx(e, NB, *bufA)
        wait_idx(e, NB, *bufB)
        pos = 0
        first = True
        while left > 0:
            na = min(left, NB)
            nb2 = min(left - na, NB)
            if first:
                p, q = bufA, bufB
            else:
                p, q = bufC, bufD
                load_idx(e, st0 + pos, na, *p)
                if nb2:
                    load_idx(e, st0 + pos + na, nb2, *q)
                wait_idx(e, na, *p)
                if nb2:
                    wait_idx(e, nb2, *q)
            process_batch(x, rows_of(p[0], p[1], na)
                          + rows_of(q[0], q[1], nb2))
            pos += na + nb2
            left -= na + nb2
            first = False
        if rem:
            @pl.when(wid < rem)
            def _():
                ld = load_idx(e, ch_total - rem + wid, 1, *bufC)
                ld[0].wait()
                ld[1].wait()
                process_batch(x, rows_of(sidx_c, didx_c, 1))

        if r < 2:
            load_idx(rels[r + 1][0], lo, NB, *bufA)
            load_idx(rels[r + 1][0], lo + NB, NB, *bufB)
        plsc.subcore_barrier()

        dd = pltpu.async_copy(acc_sh.at[pl.ds(s * RPT, RPT)],
                              sums.at[r, c, pl.ds(s * RPT, RPT)], dsem)
        if r == 2:
            cd = pltpu.async_copy(cnt_sh.at[pl.ds(s * RPT, RPT)],
                                  cnts.at[c, pl.ds(s * RPT, RPT)], dsem)
        else:
            lax.fori_loop(0, CHK, fill_zero_rows, 0)
        dd.wait()
        if r == 2:
            cd.wait()
        else:
            zero_acc_stripe()
            plsc.subcore_barrier()


_sc_segment_sums = pl.kernel(
    _sc_body,
    out_type=(
        jax.ShapeDtypeStruct((3, NC, N, D), jnp.float32),
        jax.ShapeDtypeStruct((NC, N, CW), jnp.float32),
    ),
    mesh=plsc.VectorSubcoreMesh(core_axis_name="c", subcore_axis_name="s"),
    compiler_params=pltpu.CompilerParams(use_tc_tiling_on_sc=False),
    scratch_types=[
        pltpu.VMEM_SHARED((N, D), jnp.float32),
        pltpu.VMEM_SHARED((N, CW), jnp.float32),
        pltpu.VMEM((CHK, D), jnp.float32),
        pltpu.VMEM((CHK, D), jnp.float32),
        pltpu.VMEM((CHK, D), jnp.float32),
        pltpu.VMEM((CHK, D), jnp.float32),
        pltpu.VMEM((NB, CHK), jnp.int32),
        pltpu.VMEM((NB, CHK), jnp.int32),
        pltpu.VMEM((NB, CHK), jnp.int32),
        pltpu.VMEM((NB, CHK), jnp.int32),
        pltpu.VMEM((NB, CHK), jnp.int32),
        pltpu.VMEM((NB, CHK), jnp.int32),
        pltpu.VMEM((NB, CHK), jnp.int32),
        pltpu.VMEM((NB, CHK), jnp.int32),
        pltpu.VMEM((CHK, CW), jnp.float32),
    ] + [pltpu.SemaphoreType.DMA] * 14,
)


BL = 1000


def _tc_body(sums, cnts, xi, Wl, bl, Wr, W1, b1, W2, b2, out):
    x_dst = xi[...]
    pre = jnp.zeros((BL, D), jnp.float32)
    for r in range(3):
        ssum = sums[r, 0] + sums[r, 1]
        cnt = cnts[0, :, r:r + 1] + cnts[1, :, r:r + 1]
        mean = ssum / jnp.maximum(cnt, 1.0)
        o = lax.dot_general(mean, Wl[r], (((1,), (1,)), ((), ())),
                            preferred_element_type=jnp.float32)
        o = o + bl[r] + lax.dot_general(x_dst, Wr[r], (((1,), (1,)), ((), ())),
                                        preferred_element_type=jnp.float32)
        pre = pre + lax.dot_general(o, W1[:, r * D:(r + 1) * D],
                                    (((1,), (1,)), ((), ())),
                                    preferred_element_type=jnp.float32)
    h = jnp.maximum(pre + b1[...], 0.0)
    out[...] = lax.dot_general(h, W2[...], (((1,), (1,)), ((), ())),
                               preferred_element_type=jnp.float32) + b2[...]


def _tc_tail(sums, cnts, x_item, Wl, bl, Wr, W1, b1, W2, b2):
    grid = N // BL
    return pl.pallas_call(
        _tc_body,
        grid=(grid,),
        in_specs=[
            pl.BlockSpec((3, NC, BL, D), lambda i: (0, 0, i, 0)),
            pl.BlockSpec((NC, BL, CW), lambda i: (0, i, 0)),
            pl.BlockSpec((BL, D), lambda i: (i, 0)),
            pl.BlockSpec((3, D, D), lambda i: (0, 0, 0)),
            pl.BlockSpec((3, 1, D), lambda i: (0, 0, 0)),
            pl.BlockSpec((3, D, D), lambda i: (0, 0, 0)),
            pl.BlockSpec((D, 3 * D), lambda i: (0, 0)),
            pl.BlockSpec((1, D), lambda i: (0, 0)),
            pl.BlockSpec((D, D), lambda i: (0, 0)),
            pl.BlockSpec((1, D), lambda i: (0, 0)),
        ],
        out_specs=pl.BlockSpec((BL, D), lambda i: (i, 0)),
        out_shape=jax.ShapeDtypeStruct((N, D), jnp.float32),
    )(sums, cnts, x_item, Wl, bl, Wr, W1, b1, W2, b2)


def kernel(x_user, x_item, ei_buys, ei_similar, ei_views,
           Wl_buys, bl_buys, Wr_buys,
           Wl_similar, bl_similar, Wr_similar,
           Wl_views, bl_views, Wr_views,
           W1, b1, W2, b2):
    E = ei_buys.shape[1]
    e0 = ei_buys.reshape(2, E // CHK, CHK)
    e1 = ei_similar.reshape(2, E // CHK, CHK)
    e2 = ei_views.reshape(2, E // CHK, CHK)
    cpat = (jnp.array([0, 1, 2, -1])[:, None, None] == jnp.arange(CW)[None, None, :])
    cpat = jnp.broadcast_to(cpat, (4, CHK, CW)).astype(jnp.float32)
    sums, cnts = _sc_segment_sums(x_user, x_item, e0, e1, e2, cpat)
    Wl = jnp.stack([Wl_buys, Wl_similar, Wl_views])
    bl = jnp.stack([bl_buys, bl_similar, bl_views])[:, None, :]
    Wr = jnp.stack([Wr_buys, Wr_similar, Wr_views])
    return _tc_tail(sums, cnts, x_item, Wl, bl, Wr,
                    W1, b1[None, :], W2, b2[None, :])


# speedup vs baseline: 13.7403x; 1.0079x over previous
import jax
import jax.numpy as jnp
from jax import lax
from jax.experimental import pallas as pl
from jax.experimental.pallas import tpu as pltpu
from jax.experimental.pallas import tpu_sc as plsc

N = 10000
D = 128
CHK = 64
NC = 2
NS = 16
NW = NC * NS
RPT = N // NS
CZR = 25
CW = 8
NB = 16
NBUF = 4
LOOK = 3


def _sc_body(xu, xi, e0, e1, e2, cpat, sums, cnts,
             acc_sh, cnt_sh, rows0, rows1, rows2, rows3,
             sidx_a, didx_a, sidx_b, didx_b,
             sidx_c, didx_c, sidx_d, didx_d,
             ones_r, gsem0, gsem1, gsem2, gsem3,
             ssem0, ssem1, ssem2, ssem3, osem,
             isem_a, isem_b, isem_c, isem_d, dsem):
    c = lax.axis_index("c")
    s = lax.axis_index("s")
    wid = s * NC + c
    ch_total = e0.shape[1]
    base = ch_total // NW
    rem = ch_total % NW
    lo = wid * base
    rows_bufs = (rows0, rows1, rows2, rows3)
    gsems = (gsem0, gsem1, gsem2, gsem3)
    ssems = (ssem0, ssem1, ssem2, ssem3)

    z16 = jnp.zeros((16,), jnp.float32)

    def fill_zero_rows(i, carry):
        for q in range(D // 16):
            rows0[i, pl.ds(q * 16, 16)] = z16
        return carry

    def load_idx(e, start, n, si, di, isem):
        a = pltpu.async_copy(e.at[0, pl.ds(start, n), :], si.at[pl.ds(0, n)], isem)
        b = pltpu.async_copy(e.at[1, pl.ds(start, n), :], di.at[pl.ds(0, n)], isem)
        return a, b

    def wait_idx(e, n, si, di, isem):
        pltpu.make_async_copy(e.at[0, pl.ds(0, n), :], si.at[pl.ds(0, n)], isem).wait()
        pltpu.make_async_copy(e.at[1, pl.ds(0, n), :], di.at[pl.ds(0, n)], isem).wait()

    def process_batch(x, chunks):
        nchunk = len(chunks)
        gd = [None] * NBUF
        sd = [None] * NBUF
        od = []
        for t in range(min(LOOK, nchunk)):
            gd[t] = pltpu.async_copy(x.at[chunks[t][0]], rows_bufs[t], gsems[t])
        for j in range(nchunk):
            t = j + LOOK
            if t < nchunk:
                tb = t % NBUF
                if sd[tb] is not None:
                    sd[tb].wait()
                gd[tb] = pltpu.async_copy(x.at[chunks[t][0]],
                                          rows_bufs[tb], gsems[tb])
            b = j % NBUF
            gd[b].wait()
            sd[b] = pltpu.async_copy(rows_bufs[b], acc_sh.at[chunks[j][1]],
                                     ssems[b], add=True)
            od.append(pltpu.async_copy(ones_r, cnt_sh.at[chunks[j][1]],
                                       osem, add=True))
        for dsc in sd:
            if dsc is not None:
                dsc.wait()
        for dsc in od:
            dsc.wait()

    def rows_of(si, di, n):
        return [(si.at[j], di.at[j]) for j in range(n)]

    bufA = (sidx_a, didx_a, isem_a)
    bufB = (sidx_b, didx_b, isem_b)
    bufC = (sidx_c, didx_c, isem_c)
    bufD = (sidx_d, didx_d, isem_d)

    def zero_acc_stripe():
        for j in range(RPT // CHK):
            pltpu.sync_copy(rows0, acc_sh.at[pl.ds(s * RPT + j * CHK, CHK)])
        tail = RPT % CHK
        if tail:
            pltpu.sync_copy(rows0.at[pl.ds(0, tail)],
                            acc_sh.at[pl.ds(s * RPT + (RPT // CHK) * CHK, tail)])

    rels = ((e0, xu), (e1, xi), (e2, xu))

    load_idx(rels[0][0], lo, NB, *bufA)
    load_idx(rels[0][0], lo + NB, NB, *bufB)
    lax.fori_loop(0, CHK, fill_zero_rows, 0)
    zero_acc_stripe()
    pltpu.sync_copy(cpat.at[3], ones_r)
    for j in range(RPT // CZR):
        pltpu.sync_copy(ones_r.at[pl.ds(0, CZR)],
                        cnt_sh.at[pl.ds(s * RPT + j * CZR, CZR)])
    plsc.subcore_barrier()

    for r, (e, x) in enumerate(rels):
        pltpu.sync_copy(cpat.at[r], ones_r)

        nquad = base // (4 * NB)

        cap = ch_total - NB

        def quad_body(k, carry):
            st = lo + k * (4 * NB)
            load_idx(e, jnp.minimum(st + 2 * NB, cap), NB, *bufC)
            load_idx(e, jnp.minimum(st + 3 * NB, cap), NB, *bufD)
            wait_idx(e, NB, *bufA)
            wait_idx(e, NB, *bufB)
            process_batch(x, rows_of(sidx_a, didx_a, NB)
                          + rows_of(sidx_b, didx_b, NB))
            load_idx(e, jnp.minimum(st + 4 * NB, cap), NB, *bufA)
            load_idx(e, jnp.minimum(st + 5 * NB, cap), NB, *bufB)
            wait_idx(e, NB, *bufC)
            wait_idx(e, NB, *bufD)
            process_batch(x, rows_of(sidx_c, didx_c, NB)
                          + rows_of(sidx_d, didx_d, NB))
            return carry
        lax.fori_loop(0, nquad, quad_body, 0)

        left = base - nquad * 4 * NB
        st0 = lo + nquad * 4 * NB
        wait_idx(e, NB, *bufA)
        wait_idx(e, NB, *bufB)
        pos = 0
        first = True
        while left > 0:
            na = min(left, NB)
            nb2 = min(left - na, NB)
            if first:
                p, q = bufA, bufB
            else:
                p, q = bufC, bufD
                load_idx(e, st0 + pos, na, *p)
                if nb2:
                    load_idx(e, st0 + pos + na, nb2, *q)
                wait_idx(e, na, *p)
                if nb2:
                    wait_idx(e, nb2, *q)
            process_batch(x, rows_of(p[0], p[1], na)
                          + rows_of(q[0], q[1], nb2))
            pos += na + nb2
            left -= na + nb2
            first = False
        if rem:
            @pl.when(wid < rem)
            def _():
                ld = load_idx(e, ch_total - rem + wid, 1, *bufC)
                ld[0].wait()
                ld[1].wait()
                process_batch(x, rows_of(sidx_c, didx_c, 1))

        if r < 2:
            load_idx(rels[r + 1][0], lo, NB, *bufA)
            load_idx(rels[r + 1][0], lo + NB, NB, *bufB)
        plsc.subcore_barrier()

        dd = pltpu.async_copy(acc_sh.at[pl.ds(s * RPT, RPT)],
                              sums.at[r, c, pl.ds(s * RPT, RPT)], dsem)
        if r == 2:
            cd = pltpu.async_copy(cnt_sh.at[pl.ds(s * RPT, RPT)],
                                  cnts.at[c, pl.ds(s * RPT, RPT)], dsem)
        else:
            lax.fori_loop(0, CHK, fill_zero_rows, 0)
        dd.wait()
        if r == 2:
            cd.wait()
        else:
            zero_acc_stripe()
            plsc.subcore_barrier()


_sc_segment_sums = pl.kernel(
    _sc_body,
    out_type=(
        jax.ShapeDtypeStruct((3, NC, N, D), jnp.float32),
        jax.ShapeDtypeStruct((NC, N, CW), jnp.float32),
    ),
    mesh=plsc.VectorSubcoreMesh(core_axis_name="c", subcore_axis_name="s"),
    compiler_params=pltpu.CompilerParams(use_tc_tiling_on_sc=False),
    scratch_types=[
        pltpu.VMEM_SHARED((N, D), jnp.float32),
        pltpu.VMEM_SHARED((N, CW), jnp.float32),
        pltpu.VMEM((CHK, D), jnp.float32),
        pltpu.VMEM((CHK, D), jnp.float32),
        pltpu.VMEM((CHK, D), jnp.float32),
        pltpu.VMEM((CHK, D), jnp.float32),
        pltpu.VMEM((NB, CHK), jnp.int32),
        pltpu.VMEM((NB, CHK), jnp.int32),
        pltpu.VMEM((NB, CHK), jnp.int32),
        pltpu.VMEM((NB, CHK), jnp.int32),
        pltpu.VMEM((NB, CHK), jnp.int32),
        pltpu.VMEM((NB, CHK), jnp.int32),
        pltpu.VMEM((NB, CHK), jnp.int32),
        pltpu.VMEM((NB, CHK), jnp.int32),
        pltpu.VMEM((CHK, CW), jnp.float32),
    ] + [pltpu.SemaphoreType.DMA] * 14,
)


BL = 1000


def _tc_body(sums, cnts, xi, Wl, bl, Wr, W1, b1, W2, b2, out):
    x_dst = xi[...]
    pre = jnp.zeros((BL, D), jnp.float32)
    for r in range(3):
        ssum = sums[r, 0] + sums[r, 1]
        cnt = cnts[0, :, r:r + 1] + cnts[1, :, r:r + 1]
        mean = ssum / jnp.maximum(cnt, 1.0)
        o = lax.dot_general(mean, Wl[r], (((1,), (1,)), ((), ())),
                            preferred_element_type=jnp.float32)
        o = o + bl[r] + lax.dot_general(x_dst, Wr[r], (((1,), (1,)), ((), ())),
                                        preferred_element_type=jnp.float32)
        pre = pre + lax.dot_general(o, W1[:, r * D:(r + 1) * D],
                                    (((1,), (1,)), ((), ())),
                                    preferred_element_type=jnp.float32)
    h = jnp.maximum(pre + b1[...], 0.0)
    out[...] = lax.dot_general(h, W2[...], (((1,), (1,)), ((), ())),
                               preferred_element_type=jnp.float32) + b2[...]


def _tc_tail(sums, cnts, x_item, Wl, bl, Wr, W1, b1, W2, b2):
    grid = N // BL
    return pl.pallas_call(
        _tc_body,
        grid=(grid,),
        in_specs=[
            pl.BlockSpec((3, NC, BL, D), lambda i: (0, 0, i, 0)),
            pl.BlockSpec((NC, BL, CW), lambda i: (0, i, 0)),
            pl.BlockSpec((BL, D), lambda i: (i, 0)),
            pl.BlockSpec((3, D, D), lambda i: (0, 0, 0)),
            pl.BlockSpec((3, 1, D), lambda i: (0, 0, 0)),
            pl.BlockSpec((3, D, D), lambda i: (0, 0, 0)),
            pl.BlockSpec((D, 3 * D), lambda i: (0, 0)),
            pl.BlockSpec((1, D), lambda i: (0, 0)),
            pl.BlockSpec((D, D), lambda i: (0, 0)),
            pl.BlockSpec((1, D), lambda i: (0, 0)),
        ],
        out_specs=pl.BlockSpec((BL, D), lambda i: (i, 0)),
        out_shape=jax.ShapeDtypeStruct((N, D), jnp.float32),
    )(sums, cnts, x_item, Wl, bl, Wr, W1, b1, W2, b2)


def kernel(x_user, x_item, ei_buys, ei_similar, ei_views,
           Wl_buys, bl_buys, Wr_buys,
           Wl_similar, bl_similar, Wr_similar,
           Wl_views, bl_views, Wr_views,
           W1, b1, W2, b2):
    E = ei_buys.shape[1]
    e0 = ei_buys.reshape(2, E // CHK, CHK)
    e1 = ei_similar.reshape(2, E // CHK, CHK)
    e2 = ei_views.reshape(2, E // CHK, CHK)
    cpat = (jnp.array([0, 1, 2, -1])[:, None, None] == jnp.arange(CW)[None, None, :])
    cpat = jnp.broadcast_to(cpat, (4, CHK, CW)).astype(jnp.float32)
    sums, cnts = _sc_segment_sums(x_user, x_item, e0, e1, e2, cpat)
    Wl = jnp.stack([Wl_buys, Wl_similar, Wl_views])
    bl = jnp.stack([bl_buys, bl_similar, bl_views])[:, None, :]
    Wr = jnp.stack([Wr_buys, Wr_similar, Wr_views])
    return _tc_tail(sums, cnts, x_item, Wl, bl, Wr,
                    W1, b1[None, :], W2, b2[None, :])
